# baseline (speedup 1.0000x reference)
import functools

import jax
import jax.numpy as jnp
from jax import lax
from jax.experimental import pallas as pl
from jax.experimental.pallas import tpu as pltpu
from jax.experimental.pallas import tpu_sc as plsc

_N_ATOMS = 320000
_N_BASIS = 128
_N_MOL = 10000

_BLK = 6400


def _mv_body(x_ref, w_ref, b_ref, y_ref):
    y_ref[...] = (
        jnp.sum(x_ref[...] * w_ref[...], axis=1, keepdims=True) + b_ref[...]
    )


def _matvec(x, W, b2d):
    n = x.shape[0]
    return pl.pallas_call(
        _mv_body,
        grid=(n // _BLK,),
        in_specs=[
            pl.BlockSpec((_BLK, _N_BASIS), lambda i: (i, 0)),
            pl.BlockSpec((1, _N_BASIS), lambda i: (0, 0)),
            pl.BlockSpec((1, 1), lambda i: (0, 0)),
        ],
        out_specs=pl.BlockSpec((_BLK, 1), lambda i: (i, 0)),
        out_shape=jax.ShapeDtypeStruct((n, 1), jnp.float32),
    )(x, W, b2d)


_NSUB = 16
_CHUNK = _N_ATOMS // _NSUB
_ROWS = 160
_CPAD = _ROWS * 128
_GRP = 8
_HALF = _N_MOL // 2
_ACC = 5120
_SLC = _ACC // _NSUB
_TRASH = _ACC - 1


def _seg_body(y_hbm, idx_hbm, out_hbm, y_v, idx_v, vals_v, lidx_v, z_v, sem, acc_sh):
    c = lax.axis_index("c")
    s = lax.axis_index("s")
    base = s * _CHUNK

    cp_y = pltpu.make_async_copy(
        y_hbm.at[pl.ds(base, _CHUNK)], y_v.at[pl.ds(0, _CHUNK)], sem)
    cp_y.start()
    cp_i = pltpu.make_async_copy(
        idx_hbm.at[pl.ds(base, _CHUNK)], idx_v.at[pl.ds(0, _CHUNK)], sem)
    cp_i.start()

    zf = jnp.zeros((16,), jnp.float32)
    pad_i = jnp.full((16,), _N_MOL, jnp.int32)
    for i in range(_SLC // 16):
        z_v[pl.ds(i * 16, 16)] = zf
    pltpu.sync_copy(z_v, acc_sh.at[pl.ds(s * _SLC, _SLC)])
    for i in range((_CPAD - _CHUNK) // 16):
        y_v[pl.ds(_CHUNK + i * 16, 16)] = zf
        idx_v[pl.ds(_CHUNK + i * 16, 16)] = pad_i

    cp_i.wait()
    cp_y.wait()
    lo = c * _HALF

    def row_body(j, carry):
        for k2 in range(8):
            off = j * 128 + k2 * 16
            li = idx_v[pl.ds(off, 16)] - lo
            ok = (li >= 0) & (li < _HALF)
            lidx_v[j, pl.ds(k2 * 16, 16)] = jnp.where(ok, li, _TRASH)
            vals_v[j, pl.ds(k2 * 16, 16)] = y_v[pl.ds(off, 16)]
        return carry

    lax.fori_loop(0, _ROWS, row_body, 0)

    plsc.subcore_barrier()

    def sc_body(g, carry):
        descs = []
        for bq in range(_GRP):
            j = g * _GRP + bq
            descs.append(pltpu.async_copy(
                vals_v.at[j],
                acc_sh.at[lidx_v.at[j]],
                sem, add=True))
        for d in descs:
            d.wait()
        return carry

    lax.fori_loop(0, _ROWS // _GRP, sc_body, 0)

    plsc.subcore_barrier()

    pltpu.sync_copy(acc_sh.at[pl.ds(s * _SLC, _SLC)], z_v)
    pltpu.sync_copy(z_v, out_hbm.at[pl.ds(c * _ACC + s * _SLC, _SLC)])


def _segsum(y, idx):
    k = pl.kernel(
        _seg_body,
        mesh=plsc.VectorSubcoreMesh(core_axis_name="c", subcore_axis_name="s"),
        out_type=jax.ShapeDtypeStruct((2 * _ACC,), jnp.float32),
        scratch_types=[
            pltpu.VMEM((_CPAD,), jnp.float32),
            pltpu.VMEM((_CPAD,), jnp.int32),
            pltpu.VMEM((_ROWS, 128), jnp.float32),
            pltpu.VMEM((_ROWS, 128), jnp.int32),
            pltpu.VMEM((_SLC,), jnp.float32),
            pltpu.SemaphoreType.DMA,
            pltpu.VMEM_SHARED((_ACC,), jnp.float32),
        ],
    )
    return k(y, idx)


def kernel(x, atomic_subsystem_indices, W, b):
    idx = atomic_subsystem_indices.astype(jnp.int32)
    y = _matvec(x, W, b.reshape(1, 1))
    part = _segsum(y.reshape(-1), idx)
    return part.reshape(2, _ACC)[:, :_HALF].reshape(_N_MOL, 1)


# speedup vs baseline: 1.8989x; 1.8989x over previous
import functools

import jax
import jax.numpy as jnp
from jax import lax
from jax.experimental import pallas as pl
from jax.experimental.pallas import tpu as pltpu
from jax.experimental.pallas import tpu_sc as plsc

_N_ATOMS = 320000
_N_BASIS = 128
_N_MOL = 10000

_BLK = 6400


def _mv_body(x_ref, w_ref, b_ref, y_ref):
    y_ref[...] = (
        jnp.sum(x_ref[...] * w_ref[...], axis=1, keepdims=True) + b_ref[...]
    )


def _matvec(x, W, b2d):
    n = x.shape[0]
    return pl.pallas_call(
        _mv_body,
        grid=(n // _BLK,),
        in_specs=[
            pl.BlockSpec((_BLK, _N_BASIS), lambda i: (i, 0)),
            pl.BlockSpec((1, _N_BASIS), lambda i: (0, 0)),
            pl.BlockSpec((1, 1), lambda i: (0, 0)),
        ],
        out_specs=pl.BlockSpec((_BLK, 1), lambda i: (i, 0)),
        out_shape=jax.ShapeDtypeStruct((n, 1), jnp.float32),
    )(x, W, b2d)


_NSUB = 16
_CHUNK = _N_ATOMS // _NSUB
_CPAD = 20480
_RPAD = _CPAD + 16
_EMAX = _N_MOL + 272
_SROWS = (_EMAX // 128) + 1
_ACC = 10240
_SLC = _ACC // _NSUB
_PADBIN = 10200
_TRASH = _ACC - 1


def _seg_body(y_hbm, idx_hbm, out_hbm, y_v, idx_v, e_v, ridx_v,
              vals_v, lidx_v, z_v, sem, acc_sh):
    s = lax.axis_index("s")
    base = s * _CHUNK

    cp_y = pltpu.make_async_copy(
        y_hbm.at[pl.ds(base, _CHUNK)], y_v.at[pl.ds(0, _CHUNK)], sem)
    cp_y.start()
    cp_i = pltpu.make_async_copy(
        idx_hbm.at[pl.ds(base, _CHUNK)], idx_v.at[pl.ds(0, _CHUNK)], sem)
    cp_i.start()

    zf = jnp.zeros((16,), jnp.float32)
    pad_i = jnp.full((16,), _PADBIN, jnp.int32)
    for i in range(_SLC // 16):
        z_v[pl.ds(i * 16, 16)] = zf
    pltpu.sync_copy(z_v, acc_sh.at[pl.ds(s * _SLC, _SLC)])
    for i in range((_CPAD - _CHUNK) // 16):
        y_v[pl.ds(_CHUNK + i * 16, 16)] = zf
    for i in range((_RPAD - _CHUNK) // 16):
        idx_v[pl.ds(_CHUNK + i * 16, 16)] = pad_i
    e_v[pl.ds(0, 16)] = zf

    cp_i.wait()
    cp_y.wait()

    def p1_body(i, carry):
        g, cnt = carry
        off = i * 16
        y16 = y_v[pl.ds(off, 16)]
        i16 = idx_v[pl.ds(off, 16)]
        inx = idx_v[pl.ds(off + 1, 16)]
        e16 = plsc.cumsum(y16) + g
        m = i16 != inx
        plsc.store_compressed(e_v.at[pl.ds(16 + cnt, 16)], e16, mask=m)
        plsc.store_compressed(ridx_v.at[pl.ds(cnt, 16)], i16, mask=m)
        g = g + jnp.sum(y16, axis=0)
        cnt = cnt + jnp.sum(m.astype(jnp.int32), axis=0)
        return g, cnt

    _, cnt = lax.fori_loop(0, _CPAD // 16, p1_body, (jnp.float32(0.0),
                                                     jnp.int32(0)))

    nrows = (cnt + 127) // 128
    lane = lax.iota(jnp.int32, 16)

    def p2_body(j, carry):
        for k2 in range(8):
            off = j * 128 + k2 * 16
            ok = (off + lane) < cnt
            dv = e_v[pl.ds(16 + off, 16)] - e_v[pl.ds(15 + off, 16)]
            ri = ridx_v[pl.ds(off, 16)]
            ri = jnp.minimum(jnp.maximum(ri, 0), _ACC - 1)
            vals_v[j, pl.ds(k2 * 16, 16)] = jnp.where(ok, dv, 0.0)
            lidx_v[j, pl.ds(k2 * 16, 16)] = jnp.where(ok, ri, _TRASH)
        return carry

    lax.fori_loop(0, nrows, p2_body, 0)

    plsc.subcore_barrier()

    def sc_body(j, carry):
        pltpu.sync_copy(vals_v.at[j], acc_sh.at[lidx_v.at[j]], add=True)
        return carry

    lax.fori_loop(0, nrows, sc_body, 0)

    plsc.subcore_barrier()

    pltpu.sync_copy(acc_sh.at[pl.ds(s * _SLC, _SLC)], z_v)
    pltpu.sync_copy(z_v, out_hbm.at[pl.ds(s * _SLC, _SLC)])


def _segsum(y, idx):
    k = pl.kernel(
        _seg_body,
        mesh=plsc.VectorSubcoreMesh(
            core_axis_name="c", subcore_axis_name="s", num_cores=1),
        compiler_params=pltpu.CompilerParams(needs_layout_passes=False),
        out_type=jax.ShapeDtypeStruct((_ACC,), jnp.float32),
        scratch_types=[
            pltpu.VMEM((_CPAD,), jnp.float32),
            pltpu.VMEM((_RPAD,), jnp.int32),
            pltpu.VMEM((16 + _EMAX,), jnp.float32),
            pltpu.VMEM((_EMAX,), jnp.int32),
            pltpu.VMEM((_SROWS, 128), jnp.float32),
            pltpu.VMEM((_SROWS, 128), jnp.int32),
            pltpu.VMEM((_SLC,), jnp.float32),
            pltpu.SemaphoreType.DMA,
            pltpu.VMEM_SHARED((_ACC,), jnp.float32),
        ],
    )
    return k(y, idx)


def kernel(x, atomic_subsystem_indices, W, b):
    idx = atomic_subsystem_indices.astype(jnp.int32)
    y = _matvec(x, W, b.reshape(1, 1))
    part = _segsum(y.reshape(-1), idx)
    return part[:_N_MOL].reshape(_N_MOL, 1)


# speedup vs baseline: 3.2037x; 1.6871x over previous
import functools

import jax
import jax.numpy as jnp
from jax import lax
from jax.experimental import pallas as pl
from jax.experimental.pallas import tpu as pltpu
from jax.experimental.pallas import tpu_sc as plsc

_N_ATOMS = 320000
_N_BASIS = 128
_N_MOL = 10000

_BLK = 6400
_NGRID = _N_ATOMS // _BLK
_YPAD = _N_ATOMS


def _mv_body(x_ref, wt_ref, b_ref, y_ref):
    i = pl.program_id(0)
    xt = jnp.transpose(x_ref[...].reshape(_BLK // 128, 128, 128), (0, 2, 1))
    s = jnp.sum(xt * wt_ref[...].reshape(1, 128, 1), axis=1) + b_ref[0, 0]
    y_ref[pl.ds(i * _BLK, _BLK)] = s.reshape(_BLK)


def _matvec(x, Wt, b2d):
    n = x.shape[0]
    return pl.pallas_call(
        _mv_body,
        grid=(_NGRID,),
        in_specs=[
            pl.BlockSpec((_BLK, _N_BASIS), lambda i: (i, 0)),
            pl.BlockSpec((_N_BASIS, 1), lambda i: (0, 0)),
            pl.BlockSpec((1, 1), lambda i: (0, 0)),
        ],
        out_specs=pl.BlockSpec((_YPAD,), lambda i: (0,)),
        out_shape=jax.ShapeDtypeStruct((_YPAD,), jnp.float32),
    )(x, Wt, b2d)


_NSUB = 16
_CHUNK = _N_ATOMS // _NSUB
_CPAD = 20480
_RPAD = _CPAD + 16
_EMAX = _N_MOL + 272
_SROWS = (_EMAX // 128) + 1
_ACC = 10240
_SLC = _ACC // _NSUB
_PADBIN = 10200
_TRASH = _ACC - 1


def _seg_body(y_hbm, idx_hbm, out_hbm, y_v, idx_v, e_v, ridx_v,
              vals_v, lidx_v, z_v, sem, acc_sh):
    s = lax.axis_index("s")
    base = s * _CHUNK

    cp_y = pltpu.make_async_copy(
        y_hbm.at[pl.ds(base, _CHUNK)], y_v.at[pl.ds(0, _CHUNK)], sem)
    cp_y.start()
    cp_i = pltpu.make_async_copy(
        idx_hbm.at[pl.ds(base, _CHUNK)], idx_v.at[pl.ds(0, _CHUNK)], sem)
    cp_i.start()

    zf = jnp.zeros((16,), jnp.float32)
    pad_i = jnp.full((16,), _PADBIN, jnp.int32)
    for i in range(_SLC // 16):
        z_v[pl.ds(i * 16, 16)] = zf
    pltpu.sync_copy(z_v, acc_sh.at[pl.ds(s * _SLC, _SLC)])
    for i in range((_CPAD - _CHUNK) // 16):
        y_v[pl.ds(_CHUNK + i * 16, 16)] = zf
    for i in range((_RPAD - _CHUNK) // 16):
        idx_v[pl.ds(_CHUNK + i * 16, 16)] = pad_i
    e_v[pl.ds(0, 16)] = zf

    cp_i.wait()
    cp_y.wait()

    def p1_body(i, carry):
        g, cnt = carry
        off = i * 16
        y16 = y_v[pl.ds(off, 16)]
        i16 = idx_v[pl.ds(off, 16)]
        inx = idx_v[pl.ds(off + 1, 16)]
        e16 = plsc.cumsum(y16) + g
        m = i16 != inx
        plsc.store_compressed(e_v.at[pl.ds(16 + cnt, 16)], e16, mask=m)
        plsc.store_compressed(ridx_v.at[pl.ds(cnt, 16)], i16, mask=m)
        g = g + jnp.sum(y16, axis=0)
        cnt = cnt + jnp.sum(m.astype(jnp.int32), axis=0)
        return g, cnt

    _, cnt = lax.fori_loop(0, _CPAD // 16, p1_body, (jnp.float32(0.0),
                                                     jnp.int32(0)))

    nrows = (cnt + 127) // 128
    lane = lax.iota(jnp.int32, 16)

    def p2_body(j, carry):
        for k2 in range(8):
            off = j * 128 + k2 * 16
            ok = (off + lane) < cnt
            dv = e_v[pl.ds(16 + off, 16)] - e_v[pl.ds(15 + off, 16)]
            ri = ridx_v[pl.ds(off, 16)]
            ri = jnp.minimum(jnp.maximum(ri, 0), _ACC - 1)
            vals_v[j, pl.ds(k2 * 16, 16)] = jnp.where(ok, dv, 0.0)
            lidx_v[j, pl.ds(k2 * 16, 16)] = jnp.where(ok, ri, _TRASH)
        return carry

    lax.fori_loop(0, nrows, p2_body, 0)

    plsc.subcore_barrier()

    def sc_body(j, carry):
        pltpu.sync_copy(vals_v.at[j], acc_sh.at[lidx_v.at[j]], add=True)
        return carry

    lax.fori_loop(0, nrows, sc_body, 0)

    plsc.subcore_barrier()

    pltpu.sync_copy(acc_sh.at[pl.ds(s * _SLC, _SLC)], z_v)
    pltpu.sync_copy(z_v, out_hbm.at[pl.ds(s * _SLC, _SLC)])


def _segsum(y, idx):
    k = pl.kernel(
        _seg_body,
        mesh=plsc.VectorSubcoreMesh(
            core_axis_name="c", subcore_axis_name="s", num_cores=1),
        compiler_params=pltpu.CompilerParams(needs_layout_passes=False),
        out_type=jax.ShapeDtypeStruct((_ACC,), jnp.float32),
        scratch_types=[
            pltpu.VMEM((_CPAD,), jnp.float32),
            pltpu.VMEM((_RPAD,), jnp.int32),
            pltpu.VMEM((16 + _EMAX,), jnp.float32),
            pltpu.VMEM((_EMAX,), jnp.int32),
            pltpu.VMEM((_SROWS, 128), jnp.float32),
            pltpu.VMEM((_SROWS, 128), jnp.int32),
            pltpu.VMEM((_SLC,), jnp.float32),
            pltpu.SemaphoreType.DMA,
            pltpu.VMEM_SHARED((_ACC,), jnp.float32),
        ],
    )
    return k(y, idx)


def kernel(x, atomic_subsystem_indices, W, b):
    idx = atomic_subsystem_indices.astype(jnp.int32)
    y = _matvec(x, W.reshape(_N_BASIS, 1), b.reshape(1, 1))
    part = _segsum(y, idx)
    return part[:_N_MOL].reshape(_N_MOL, 1)


# speedup vs baseline: 3.5958x; 1.1224x over previous
import functools

import jax
import jax.numpy as jnp
from jax import lax
from jax.experimental import pallas as pl
from jax.experimental.pallas import tpu as pltpu
from jax.experimental.pallas import tpu_sc as plsc

_N_ATOMS = 320000
_N_BASIS = 128
_N_MOL = 10000

_BLK = 12800
_NGRID = _N_ATOMS // _BLK
_YPAD = _N_ATOMS


def _mv_body(x_ref, wt_ref, b_ref, y_ref):
    i = pl.program_id(0)
    xt = jnp.transpose(x_ref[...].reshape(_BLK // 128, 128, 128), (0, 2, 1))
    s = jnp.sum(xt * wt_ref[...].reshape(1, 128, 1), axis=1) + b_ref[0, 0]
    y_ref[pl.ds(i * _BLK, _BLK)] = s.reshape(_BLK)


def _matvec(x, Wt, b2d):
    n = x.shape[0]
    return pl.pallas_call(
        _mv_body,
        grid=(_NGRID,),
        in_specs=[
            pl.BlockSpec((_BLK, _N_BASIS), lambda i: (i, 0)),
            pl.BlockSpec((_N_BASIS, 1), lambda i: (0, 0)),
            pl.BlockSpec((1, 1), lambda i: (0, 0)),
        ],
        out_specs=pl.BlockSpec((_YPAD,), lambda i: (0,)),
        out_shape=jax.ShapeDtypeStruct((_YPAD,), jnp.float32),
    )(x, Wt, b2d)


_NSUB = 16
_CHUNK = _N_ATOMS // _NSUB
_CPAD = 20480
_RPAD = _CPAD + 16
_EMAX = _N_MOL + 272
_SROWS = (_EMAX // 128) + 1
_ACC = 10240
_SLC = _ACC // _NSUB
_PADBIN = 10200
_TRASH = _ACC - 1


def _seg_body(y_hbm, idx_hbm, out_hbm, y_v, idx_v, e_v, ridx_v,
              vals_v, lidx_v, z_v, sem, acc_sh):
    s = lax.axis_index("s")
    base = s * _CHUNK

    cp_y = pltpu.make_async_copy(
        y_hbm.at[pl.ds(base, _CHUNK)], y_v.at[pl.ds(0, _CHUNK)], sem)
    cp_y.start()
    cp_i = pltpu.make_async_copy(
        idx_hbm.at[pl.ds(base, _CHUNK)], idx_v.at[pl.ds(0, _CHUNK)], sem)
    cp_i.start()

    zf = jnp.zeros((16,), jnp.float32)
    pad_i = jnp.full((16,), _PADBIN, jnp.int32)
    for i in range(_SLC // 16):
        z_v[pl.ds(i * 16, 16)] = zf
    pltpu.sync_copy(z_v, acc_sh.at[pl.ds(s * _SLC, _SLC)])
    for i in range((_CPAD - _CHUNK) // 16):
        y_v[pl.ds(_CHUNK + i * 16, 16)] = zf
    for i in range((_RPAD - _CHUNK) // 16):
        idx_v[pl.ds(_CHUNK + i * 16, 16)] = pad_i
    e_v[pl.ds(0, 16)] = zf

    cp_i.wait()
    cp_y.wait()

    def p1_body(i, carry):
        g, cnt = carry
        off = i * 16
        y16 = y_v[pl.ds(off, 16)]
        i16 = idx_v[pl.ds(off, 16)]
        inx = idx_v[pl.ds(off + 1, 16)]
        e16 = plsc.cumsum(y16) + g
        m = i16 != inx
        plsc.store_compressed(e_v.at[pl.ds(16 + cnt, 16)], e16, mask=m)
        plsc.store_compressed(ridx_v.at[pl.ds(cnt, 16)], i16, mask=m)
        g = g + jnp.sum(y16, axis=0)
        cnt = cnt + jnp.sum(m.astype(jnp.int32), axis=0)
        return g, cnt

    _, cnt = lax.fori_loop(0, _CPAD // 16, p1_body, (jnp.float32(0.0),
                                                     jnp.int32(0)))

    nrows = (cnt + 127) // 128
    lane = lax.iota(jnp.int32, 16)

    def p2_body(j, carry):
        for k2 in range(8):
            off = j * 128 + k2 * 16
            ok = (off + lane) < cnt
            dv = e_v[pl.ds(16 + off, 16)] - e_v[pl.ds(15 + off, 16)]
            ri = ridx_v[pl.ds(off, 16)]
            ri = jnp.minimum(jnp.maximum(ri, 0), _ACC - 1)
            vals_v[j, pl.ds(k2 * 16, 16)] = jnp.where(ok, dv, 0.0)
            lidx_v[j, pl.ds(k2 * 16, 16)] = jnp.where(ok, ri, _TRASH)
        return carry

    lax.fori_loop(0, nrows, p2_body, 0)

    plsc.subcore_barrier()

    def sc_body(j, carry):
        pltpu.sync_copy(vals_v.at[j], acc_sh.at[lidx_v.at[j]], add=True)
        return carry

    lax.fori_loop(0, nrows, sc_body, 0)

    plsc.subcore_barrier()

    pltpu.sync_copy(acc_sh.at[pl.ds(s * _SLC, _SLC)], z_v)
    pltpu.sync_copy(z_v, out_hbm.at[pl.ds(s * _SLC, _SLC)])


def _segsum(y, idx):
    k = pl.kernel(
        _seg_body,
        mesh=plsc.VectorSubcoreMesh(
            core_axis_name="c", subcore_axis_name="s", num_cores=1),
        compiler_params=pltpu.CompilerParams(needs_layout_passes=False),
        out_type=jax.ShapeDtypeStruct((_ACC,), jnp.float32),
        scratch_types=[
            pltpu.VMEM((_CPAD,), jnp.float32),
            pltpu.VMEM((_RPAD,), jnp.int32),
            pltpu.VMEM((16 + _EMAX,), jnp.float32),
            pltpu.VMEM((_EMAX,), jnp.int32),
            pltpu.VMEM((_SROWS, 128), jnp.float32),
            pltpu.VMEM((_SROWS, 128), jnp.int32),
            pltpu.VMEM((_SLC,), jnp.float32),
            pltpu.SemaphoreType.DMA,
            pltpu.VMEM_SHARED((_ACC,), jnp.float32),
        ],
    )
    return k(y, idx)


def kernel(x, atomic_subsystem_indices, W, b):
    idx = atomic_subsystem_indices.astype(jnp.int32)
    y = _matvec(x, W.reshape(_N_BASIS, 1), b.reshape(1, 1))
    part = _segsum(y, idx)
    return part[:_N_MOL].reshape(_N_MOL, 1)


# speedup vs baseline: 3.8270x; 1.0643x over previous
import functools

import jax
import jax.numpy as jnp
from jax import lax
from jax.experimental import pallas as pl
from jax.experimental.pallas import tpu as pltpu
from jax.experimental.pallas import tpu_sc as plsc

_N_ATOMS = 320000
_N_BASIS = 128
_N_MOL = 10000

_BLK = 32000
_NGRID = _N_ATOMS // _BLK
_YPAD = _N_ATOMS


def _mv_body(x_ref, wt_ref, b_ref, y_ref):
    i = pl.program_id(0)
    xt = jnp.transpose(x_ref[...].reshape(_BLK // 128, 128, 128), (0, 2, 1))
    s = jnp.sum(xt * wt_ref[...].reshape(1, 128, 1), axis=1) + b_ref[0, 0]
    y_ref[pl.ds(i * _BLK, _BLK)] = s.reshape(_BLK)


def _matvec(x, Wt, b2d):
    n = x.shape[0]
    return pl.pallas_call(
        _mv_body,
        grid=(_NGRID,),
        in_specs=[
            pl.BlockSpec((_BLK, _N_BASIS), lambda i: (i, 0)),
            pl.BlockSpec((_N_BASIS, 1), lambda i: (0, 0)),
            pl.BlockSpec((1, 1), lambda i: (0, 0)),
        ],
        out_specs=pl.BlockSpec((_YPAD,), lambda i: (0,)),
        out_shape=jax.ShapeDtypeStruct((_YPAD,), jnp.float32),
    )(x, Wt, b2d)


_NSUB = 16
_CHUNK = _N_ATOMS // _NSUB
_CPAD = 20480
_RPAD = _CPAD + 16
_EMAX = _N_MOL + 272
_SROWS = (_EMAX // 128) + 1
_ACC = 10240
_SLC = _ACC // _NSUB
_PADBIN = 10200
_TRASH = _ACC - 1


def _seg_body(y_hbm, idx_hbm, out_hbm, y_v, idx_v, e_v, ridx_v,
              vals_v, lidx_v, z_v, sem, acc_sh):
    s = lax.axis_index("s")
    base = s * _CHUNK

    cp_y = pltpu.make_async_copy(
        y_hbm.at[pl.ds(base, _CHUNK)], y_v.at[pl.ds(0, _CHUNK)], sem)
    cp_y.start()
    cp_i = pltpu.make_async_copy(
        idx_hbm.at[pl.ds(base, _CHUNK)], idx_v.at[pl.ds(0, _CHUNK)], sem)
    cp_i.start()

    zf = jnp.zeros((16,), jnp.float32)
    pad_i = jnp.full((16,), _PADBIN, jnp.int32)
    for i in range(_SLC // 16):
        z_v[pl.ds(i * 16, 16)] = zf
    pltpu.sync_copy(z_v, acc_sh.at[pl.ds(s * _SLC, _SLC)])
    for i in range((_CPAD - _CHUNK) // 16):
        y_v[pl.ds(_CHUNK + i * 16, 16)] = zf
    for i in range((_RPAD - _CHUNK) // 16):
        idx_v[pl.ds(_CHUNK + i * 16, 16)] = pad_i
    e_v[pl.ds(0, 16)] = zf

    cp_i.wait()
    cp_y.wait()

    def p1_body(i, carry):
        g, cnt = carry
        off = i * 16
        y16 = y_v[pl.ds(off, 16)]
        i16 = idx_v[pl.ds(off, 16)]
        inx = idx_v[pl.ds(off + 1, 16)]
        e16 = plsc.cumsum(y16) + g
        m = i16 != inx
        plsc.store_compressed(e_v.at[pl.ds(16 + cnt, 16)], e16, mask=m)
        plsc.store_compressed(ridx_v.at[pl.ds(cnt, 16)], i16, mask=m)
        g = g + jnp.sum(y16, axis=0)
        cnt = cnt + jnp.sum(m.astype(jnp.int32), axis=0)
        return g, cnt

    _, cnt = lax.fori_loop(0, _CPAD // 16, p1_body, (jnp.float32(0.0),
                                                     jnp.int32(0)))

    nrows = (cnt + 127) // 128
    lane = lax.iota(jnp.int32, 16)

    def p2_body(j, carry):
        for k2 in range(8):
            off = j * 128 + k2 * 16
            ok = (off + lane) < cnt
            dv = e_v[pl.ds(16 + off, 16)] - e_v[pl.ds(15 + off, 16)]
            ri = ridx_v[pl.ds(off, 16)]
            ri = jnp.minimum(jnp.maximum(ri, 0), _ACC - 1)
            vals_v[j, pl.ds(k2 * 16, 16)] = jnp.where(ok, dv, 0.0)
            lidx_v[j, pl.ds(k2 * 16, 16)] = jnp.where(ok, ri, _TRASH)
        return carry

    lax.fori_loop(0, nrows, p2_body, 0)

    plsc.subcore_barrier()

    def sc_body(j, carry):
        pltpu.sync_copy(vals_v.at[j], acc_sh.at[lidx_v.at[j]], add=True)
        return carry

    lax.fori_loop(0, nrows, sc_body, 0)

    plsc.subcore_barrier()

    pltpu.sync_copy(acc_sh.at[pl.ds(s * _SLC, _SLC)], z_v)
    pltpu.sync_copy(z_v, out_hbm.at[pl.ds(s * _SLC, _SLC)])


def _segsum(y, idx):
    k = pl.kernel(
        _seg_body,
        mesh=plsc.VectorSubcoreMesh(
            core_axis_name="c", subcore_axis_name="s", num_cores=1),
        compiler_params=pltpu.CompilerParams(needs_layout_passes=False),
        out_type=jax.ShapeDtypeStruct((_ACC,), jnp.float32),
        scratch_types=[
            pltpu.VMEM((_CPAD,), jnp.float32),
            pltpu.VMEM((_RPAD,), jnp.int32),
            pltpu.VMEM((16 + _EMAX,), jnp.float32),
            pltpu.VMEM((_EMAX,), jnp.int32),
            pltpu.VMEM((_SROWS, 128), jnp.float32),
            pltpu.VMEM((_SROWS, 128), jnp.int32),
            pltpu.VMEM((_SLC,), jnp.float32),
            pltpu.SemaphoreType.DMA,
            pltpu.VMEM_SHARED((_ACC,), jnp.float32),
        ],
    )
    return k(y, idx)


def kernel(x, atomic_subsystem_indices, W, b):
    idx = atomic_subsystem_indices.astype(jnp.int32)
    y = _matvec(x, W.reshape(_N_BASIS, 1), b.reshape(1, 1))
    part = _segsum(y, idx)
    return part[:_N_MOL].reshape(_N_MOL, 1)
